# baseline (speedup 1.0000x reference)
import functools

import jax
import jax.numpy as jnp
from jax import lax
from jax.experimental import pallas as pl
from jax.experimental.pallas import tpu as pltpu
from jax.experimental.pallas import tpu_sc as plsc

N = 16384
K = 1000
NC = 2
NS = 16
NW = NC * NS
SPW = N // NW
CA = 104
CB_ = 96
NCHUNK = 10
L = 16

_C0 = [0, 104, 200, 304, 400, 504, 600, 704, 800, 904]
_CN = [104, 96] * 5


def _onehot_body(x_hbm, z_hbm, out_hbm, x_v, buf_a, buf_b, sem_a, sem_b):
    wid = lax.axis_index("s") * NC + lax.axis_index("c")
    base = wid * SPW

    pltpu.sync_copy(x_hbm.at[pl.ds(base, SPW)], x_v)
    za = pltpu.async_copy(z_hbm, buf_a, sem_a)
    zb = pltpu.async_copy(z_hbm.at[pl.ds(0, CB_), :], buf_b, sem_b)
    za.wait()
    zb.wait()

    zeros = jnp.zeros((L,), jnp.int32)
    ones = jnp.ones((L,), jnp.int32)
    iota = lax.iota(jnp.int32, L)

    def scan_scatter(buf, c0, cn, val):
        for j in range(SPW // L):
            xv = x_v[pl.ds(j * L, L)]
            rows = xv - c0
            mask = (xv >= c0) & (xv < c0 + cn)
            plsc.store_scatter(buf, [rows, iota + j * L], val, mask=mask)

    for chunk in range(NCHUNK):
        buf, sem = (buf_a, sem_a) if chunk % 2 == 0 else (buf_b, sem_b)
        c0, cn = _C0[chunk], _CN[chunk]
        if chunk >= 2:
            pltpu.make_async_copy(
                buf, out_hbm.at[pl.ds(0, cn), pl.ds(0, SPW)], sem
            ).wait()
            scan_scatter(buf, _C0[chunk - 2], _CN[chunk - 2], zeros)
        scan_scatter(buf, c0, cn, ones)
        pltpu.async_copy(
            buf, out_hbm.at[pl.ds(c0, cn), pl.ds(base, SPW)], sem
        )

    pltpu.make_async_copy(
        buf_a, out_hbm.at[pl.ds(0, CA), pl.ds(0, SPW)], sem_a
    ).wait()
    pltpu.make_async_copy(
        buf_b, out_hbm.at[pl.ds(0, CB_), pl.ds(0, SPW)], sem_b
    ).wait()


@jax.jit
def kernel(x):
    run = functools.partial(
        pl.kernel,
        out_type=jax.ShapeDtypeStruct((K, N), jnp.int32),
        mesh=plsc.VectorSubcoreMesh(core_axis_name="c", subcore_axis_name="s"),
        compiler_params=pltpu.CompilerParams(needs_layout_passes=False),
        scratch_types=[
            pltpu.VMEM((SPW,), jnp.int32),
            pltpu.VMEM((CA, SPW), jnp.int32),
            pltpu.VMEM((CB_, SPW), jnp.int32),
            pltpu.SemaphoreType.DMA,
            pltpu.SemaphoreType.DMA,
        ],
    )(_onehot_body)
    zeros_chunk = jnp.zeros((CA, SPW), jnp.int32)
    out_t = run(x, zeros_chunk)
    return out_t.T


# speedup vs baseline: 1.2441x; 1.2441x over previous
import functools

import jax
import jax.numpy as jnp
from jax import lax
from jax.experimental import pallas as pl
from jax.experimental.pallas import tpu as pltpu
from jax.experimental.pallas import tpu_sc as plsc

N = 16384
K = 1000
NC = 2
NS = 16
NW = NC * NS
SPW = N // NW
CC = 200
NCHUNK = K // CC
L = 16


def _onehot_body(x_hbm, z_hbm, out_hbm, x_v, buf):
    wid = lax.axis_index("s") * NC + lax.axis_index("c")
    base = wid * SPW

    pltpu.sync_copy(x_hbm.at[pl.ds(base, SPW)], x_v)
    pltpu.sync_copy(z_hbm, buf)

    zeros = jnp.zeros((L,), jnp.int32)
    ones = jnp.ones((L,), jnp.int32)
    iota = lax.iota(jnp.int32, L)
    cc_u = jnp.uint32(CC)

    for chunk in range(NCHUNK):
        c0 = chunk * CC
        for j in range(SPW // L):
            xv = x_v[pl.ds(j * L, L)]
            cols = iota + j * L
            if chunk > 0:
                prows = xv - (c0 - CC)
                pmask = plsc.bitcast(prows, jnp.uint32) < cc_u
                plsc.store_scatter(buf, [prows, cols], zeros, mask=pmask)
            rows = xv - c0
            mask = plsc.bitcast(rows, jnp.uint32) < cc_u
            plsc.store_scatter(buf, [rows, cols], ones, mask=mask)
        pltpu.sync_copy(buf, out_hbm.at[pl.ds(c0, CC), pl.ds(base, SPW)])


@jax.jit
def kernel(x):
    run = functools.partial(
        pl.kernel,
        out_type=jax.ShapeDtypeStruct((K, N), jnp.int32),
        mesh=plsc.VectorSubcoreMesh(core_axis_name="c", subcore_axis_name="s"),
        compiler_params=pltpu.CompilerParams(needs_layout_passes=False),
        scratch_types=[
            pltpu.VMEM((SPW,), jnp.int32),
            pltpu.VMEM((CC, SPW), jnp.int32),
        ],
    )(_onehot_body)
    zeros_chunk = jnp.zeros((CC, SPW), jnp.int32)
    out_t = run(x, zeros_chunk)
    return out_t.T
